# baseline (speedup 1.0000x reference)
import functools

import jax
import jax.numpy as jnp
from jax import lax
from jax.experimental import pallas as pl
from jax.experimental.pallas import tpu as pltpu
from jax.experimental.pallas import tpu_sc as plsc

PRE_SEQ_LEN = 128
HIDDEN = 1024
NUM_LAYERS = 24
OUT_DIM = NUM_LAYERS * 2 * HIDDEN
BATCH = 16
ROWS = BATCH * PRE_SEQ_LEN
BLOCK_N = 2048

NC, NS = 2, 16
NW = NC * NS
RPW = ROWS // NW
CCH = 768
NCH = OUT_DIM // CCH


def _table_kernel(emb_ref, w1_ref, b1_ref, w2_ref, b2_ref, tbl_ref, htab_ref):
    j = pl.program_id(0)

    @pl.when(j == 0)
    def _init():
        h = jnp.dot(emb_ref[...], w1_ref[...],
                    preferred_element_type=jnp.float32)
        htab_ref[...] = jnp.tanh(h + b1_ref[...]).astype(jnp.bfloat16)

    tbl_ref[...] = jnp.dot(
        htab_ref[...], w2_ref[...].astype(jnp.bfloat16),
        preferred_element_type=jnp.float32) + b2_ref[...]


def _sc_gather(tbl_hbm, ids_hbm, out_hbm, ids_v, buf0, buf1, sem0, sem1):
    wid = lax.axis_index("s") * NC + lax.axis_index("c")
    base = wid * RPW
    rows = pl.ds(base, RPW)
    pltpu.sync_copy(ids_hbm.at[rows], ids_v)

    pltpu.async_copy(tbl_hbm.at[ids_v, pl.ds(0, 1)], buf0, sem0)
    pltpu.async_copy(tbl_hbm.at[ids_v, pl.ds(1, 1)], buf1, sem1)

    def body(i, carry):
        ch0 = 2 * i
        pltpu.make_async_copy(tbl_hbm.at[ids_v, pl.ds(ch0, 1)], buf0, sem0).wait()
        pltpu.sync_copy(buf0, out_hbm.at[rows, pl.ds(ch0, 1)])

        @pl.when(ch0 + 2 < NCH)
        def _():
            pltpu.async_copy(tbl_hbm.at[ids_v, pl.ds(ch0 + 2, 1)], buf0, sem0)

        pltpu.make_async_copy(tbl_hbm.at[ids_v, pl.ds(ch0 + 1, 1)], buf1, sem1).wait()
        pltpu.sync_copy(buf1, out_hbm.at[rows, pl.ds(ch0 + 1, 1)])

        @pl.when(ch0 + 3 < NCH)
        def _():
            pltpu.async_copy(tbl_hbm.at[ids_v, pl.ds(ch0 + 3, 1)], buf1, sem1)

        return carry

    lax.fori_loop(0, NCH // 2, body, 0)


@jax.jit
def kernel(prefix, emb, W1, b1, W2, b2):
    ids = prefix.reshape(ROWS).astype(jnp.int32)
    b1r = b1.reshape(1, HIDDEN)
    b2r = b2.reshape(1, OUT_DIM)

    table = pl.pallas_call(
        _table_kernel,
        grid=(OUT_DIM // BLOCK_N,),
        in_specs=[
            pl.BlockSpec((PRE_SEQ_LEN, HIDDEN), lambda j: (0, 0)),
            pl.BlockSpec((HIDDEN, HIDDEN), lambda j: (0, 0)),
            pl.BlockSpec((1, HIDDEN), lambda j: (0, 0)),
            pl.BlockSpec((HIDDEN, BLOCK_N), lambda j: (0, j)),
            pl.BlockSpec((1, BLOCK_N), lambda j: (0, j)),
        ],
        out_specs=pl.BlockSpec((PRE_SEQ_LEN, BLOCK_N), lambda j: (0, j)),
        out_shape=jax.ShapeDtypeStruct((PRE_SEQ_LEN, OUT_DIM), jnp.float32),
        scratch_shapes=[pltpu.VMEM((PRE_SEQ_LEN, HIDDEN), jnp.bfloat16)],
        compiler_params=pltpu.CompilerParams(
            dimension_semantics=("arbitrary",),
        ),
    )(emb, W1, b1r, W2, b2r)

    tbl3 = table.reshape(PRE_SEQ_LEN, NCH, CCH)

    gather = pl.kernel(
        _sc_gather,
        out_type=jax.ShapeDtypeStruct((ROWS, NCH, CCH), jnp.float32),
        mesh=plsc.VectorSubcoreMesh(core_axis_name="c", subcore_axis_name="s"),
        scratch_types=[
            pltpu.VMEM((RPW,), jnp.int32),
            pltpu.VMEM((RPW, 1, CCH), jnp.float32),
            pltpu.VMEM((RPW, 1, CCH), jnp.float32),
            pltpu.SemaphoreType.DMA,
            pltpu.SemaphoreType.DMA,
        ],
    )
    out3 = gather(tbl3, ids)
    return out3.reshape(BATCH, PRE_SEQ_LEN, OUT_DIM)


# speedup vs baseline: 4.0190x; 4.0190x over previous
import functools

import jax
import jax.numpy as jnp
from jax.experimental import pallas as pl
from jax.experimental.pallas import tpu as pltpu

PRE_SEQ_LEN = 128
HIDDEN = 1024
NUM_LAYERS = 24
OUT_DIM = NUM_LAYERS * 2 * HIDDEN
BATCH = 16
BLOCK_N = 2048


def _fused_kernel(prefix_ref, emb_ref, w1_ref, b1_ref, w2_ref, b2_ref,
                  out_ref, htab_ref, onehot_ref):
    j = pl.program_id(0)

    @pl.when(j == 0)
    def _init():
        h = jnp.dot(emb_ref[...], w1_ref[...],
                    preferred_element_type=jnp.float32)
        htab_ref[...] = jnp.tanh(h + b1_ref[...]).astype(jnp.bfloat16)
        ids = prefix_ref[...]
        iota = jax.lax.broadcasted_iota(jnp.int32, (BATCH, PRE_SEQ_LEN, PRE_SEQ_LEN), 2)
        onehot_ref[...] = (ids[:, :, None] == iota).astype(jnp.bfloat16)

    t = jnp.dot(htab_ref[...], w2_ref[...].astype(jnp.bfloat16),
                preferred_element_type=jnp.float32).astype(jnp.bfloat16)
    out_ref[...] = jax.lax.dot_general(
        onehot_ref[...], t,
        dimension_numbers=(((2,), (0,)), ((), ())),
        preferred_element_type=jnp.float32) + b2_ref[...]


@jax.jit
def kernel(prefix, emb, W1, b1, W2, b2):
    prefix = prefix.astype(jnp.int32)
    b1r = b1.reshape(1, HIDDEN)
    b2r = b2.reshape(1, OUT_DIM)
    grid = (OUT_DIM // BLOCK_N,)
    out = pl.pallas_call(
        _fused_kernel,
        grid=grid,
        in_specs=[
            pl.BlockSpec((BATCH, PRE_SEQ_LEN), lambda j: (0, 0)),
            pl.BlockSpec((PRE_SEQ_LEN, HIDDEN), lambda j: (0, 0)),
            pl.BlockSpec((HIDDEN, HIDDEN), lambda j: (0, 0)),
            pl.BlockSpec((1, HIDDEN), lambda j: (0, 0)),
            pl.BlockSpec((HIDDEN, BLOCK_N), lambda j: (0, j)),
            pl.BlockSpec((1, BLOCK_N), lambda j: (0, j)),
        ],
        out_specs=pl.BlockSpec((BATCH, PRE_SEQ_LEN, BLOCK_N), lambda j: (0, 0, j)),
        out_shape=jax.ShapeDtypeStruct((BATCH, PRE_SEQ_LEN, OUT_DIM), jnp.float32),
        scratch_shapes=[
            pltpu.VMEM((PRE_SEQ_LEN, HIDDEN), jnp.bfloat16),
            pltpu.VMEM((BATCH, PRE_SEQ_LEN, PRE_SEQ_LEN), jnp.bfloat16),
        ],
        compiler_params=pltpu.CompilerParams(
            dimension_semantics=("arbitrary",),
        ),
    )(prefix, emb, W1, b1r, W2, b2r)
    return out


# speedup vs baseline: 4.0202x; 1.0003x over previous
import jax
import jax.numpy as jnp
from jax.experimental import pallas as pl
from jax.experimental.pallas import tpu as pltpu

PRE_SEQ_LEN = 128
HIDDEN = 1024
NUM_LAYERS = 24
OUT_DIM = NUM_LAYERS * 2 * HIDDEN
BATCH = 16
BLOCK_N = 2048
NB = OUT_DIM // BLOCK_N


def _fused_kernel(prefix_ref, emb_ref, w1_ref, b1_ref, w2_hbm, b2_ref,
                  out_hbm, htab_ref, onehot_ref,
                  w2buf0, w2buf1, obuf0, obuf1,
                  rsem0, rsem1, wsem0, wsem1):
    h = jnp.dot(emb_ref[...], w1_ref[...], preferred_element_type=jnp.float32)
    htab_ref[...] = jnp.tanh(h + b1_ref[...]).astype(jnp.bfloat16)
    ids = prefix_ref[...]
    iota = jax.lax.broadcasted_iota(
        jnp.int32, (BATCH, PRE_SEQ_LEN, PRE_SEQ_LEN), 2)
    onehot_ref[...] = (ids[:, :, None] == iota).astype(jnp.bfloat16)

    w2bufs = (w2buf0, w2buf1)
    obufs = (obuf0, obuf1)
    rsems = (rsem0, rsem1)
    wsems = (wsem0, wsem1)

    def rd(j):
        return pltpu.make_async_copy(
            w2_hbm.at[:, pl.ds(j * BLOCK_N, BLOCK_N)], w2bufs[j % 2],
            rsems[j % 2])

    def wr(j):
        return pltpu.make_async_copy(
            obufs[j % 2], out_hbm.at[:, :, pl.ds(j * BLOCK_N, BLOCK_N)],
            wsems[j % 2])

    rd(0).start()
    rd(1).start()
    for j in range(NB):
        rd(j).wait()
        if j >= 2:
            wr(j - 2).wait()
        t = jnp.dot(htab_ref[...], w2bufs[j % 2][...].astype(jnp.bfloat16),
                    preferred_element_type=jnp.float32).astype(jnp.bfloat16)
        obufs[j % 2][...] = jax.lax.dot_general(
            onehot_ref[...], t,
            dimension_numbers=(((2,), (0,)), ((), ())),
            preferred_element_type=jnp.float32
        ) + b2_ref[:, pl.ds(j * BLOCK_N, BLOCK_N)]
        wr(j).start()
        if j + 2 < NB:
            rd(j + 2).start()
    wr(NB - 2).wait()
    wr(NB - 1).wait()


@jax.jit
def kernel(prefix, emb, W1, b1, W2, b2):
    prefix = prefix.astype(jnp.int32)
    b1r = b1.reshape(1, HIDDEN)
    b2r = b2.reshape(1, OUT_DIM)
    out = pl.pallas_call(
        _fused_kernel,
        in_specs=[
            pl.BlockSpec(memory_space=pltpu.VMEM),
            pl.BlockSpec(memory_space=pltpu.VMEM),
            pl.BlockSpec(memory_space=pltpu.VMEM),
            pl.BlockSpec(memory_space=pltpu.VMEM),
            pl.BlockSpec(memory_space=pl.ANY),
            pl.BlockSpec(memory_space=pltpu.VMEM),
        ],
        out_specs=pl.BlockSpec(memory_space=pl.ANY),
        out_shape=jax.ShapeDtypeStruct((BATCH, PRE_SEQ_LEN, OUT_DIM),
                                       jnp.float32),
        scratch_shapes=[
            pltpu.VMEM((PRE_SEQ_LEN, HIDDEN), jnp.bfloat16),
            pltpu.VMEM((BATCH, PRE_SEQ_LEN, PRE_SEQ_LEN), jnp.bfloat16),
            pltpu.VMEM((HIDDEN, BLOCK_N), jnp.float32),
            pltpu.VMEM((HIDDEN, BLOCK_N), jnp.float32),
            pltpu.VMEM((BATCH, PRE_SEQ_LEN, BLOCK_N), jnp.float32),
            pltpu.VMEM((BATCH, PRE_SEQ_LEN, BLOCK_N), jnp.float32),
            pltpu.SemaphoreType.DMA,
            pltpu.SemaphoreType.DMA,
            pltpu.SemaphoreType.DMA,
            pltpu.SemaphoreType.DMA,
        ],
    )(prefix, emb, W1, b1r, W2, b2r)
    return out
